# Initial kernel scaffold; baseline (speedup 1.0000x reference)
import jax
import jax.numpy as jnp
from jax.experimental import pallas as pl


def kernel(positions, positional_embeddings):
    raise NotImplementedError("write your pallas kernel here")


# speedup vs baseline: 1.0504x; 1.0504x over previous
import functools

import jax
import jax.numpy as jnp
from jax import lax
from jax.experimental import pallas as pl
from jax.experimental.pallas import tpu as pltpu
from jax.experimental.pallas import tpu_sc as plsc

N_TOTAL = 32768
D = 1024
LANES = 16
NC, NS = 2, 16
NW = NC * NS
B_PER_W = N_TOTAL // NW
CHUNK = 64
NCHUNK = B_PER_W // CHUNK

_mesh = plsc.VectorSubcoreMesh(core_axis_name="c", subcore_axis_name="s")


_GATHER_DNUMS = lax.GatherDimensionNumbers(
    offset_dims=(), collapsed_slice_dims=(0,), start_index_map=(0,)
)


def _lane_shuffle(x, perm):
    return lax.gather(
        x, perm[:, None], _GATHER_DNUMS, (1,),
        mode=lax.GatherScatterMode.PROMISE_IN_BOUNDS,
    )


def _lane_sum(x):
    for sh in (8, 4, 2, 1):
        perm = jnp.arange(LANES, dtype=jnp.int32) ^ sh
        x = x + _lane_shuffle(x, perm)
    return x


def _rsqrt16(t):
    t = jnp.maximum(t, jnp.float32(1e-24))
    i = lax.bitcast_convert_type(t, jnp.int32)
    y = lax.bitcast_convert_type(jnp.int32(0x5F3759DF) - (i >> 1), jnp.float32)
    for _ in range(3):
        y = y * (jnp.float32(1.5) - jnp.float32(0.5) * t * y * y)
    return y


@functools.partial(
    pl.kernel,
    mesh=_mesh,
    out_type=jax.ShapeDtypeStruct((N_TOTAL, D), jnp.float32),
    scratch_types=[
        pltpu.VMEM((B_PER_W,), jnp.int32),
        pltpu.VMEM((CHUNK, D), jnp.float32),
        pltpu.SemaphoreType.DMA,
        pltpu.SemaphoreType.DMA,
    ],
)
def _gather_normalize(idx_hbm, table_hbm, out_hbm, idx_v, rows_v, gsem, osem):
    wid = lax.axis_index("s") * NC + lax.axis_index("c")
    base = wid * B_PER_W
    pltpu.sync_copy(idx_hbm.at[pl.ds(base, B_PER_W)], idx_v)

    def chunk_body(ci, _):
        pltpu.async_copy(
            table_hbm.at[idx_v.at[pl.ds(ci * CHUNK, CHUNK)]], rows_v, gsem
        ).wait()

        def row_body(r, _):
            acc = jnp.zeros((LANES,), jnp.float32)
            for j in range(D // LANES):
                s = rows_v[r, pl.ds(j * LANES, LANES)]
                acc = acc + s * s
            y = _rsqrt16(_lane_sum(acc))
            for j in range(D // LANES):
                sl = pl.ds(j * LANES, LANES)
                rows_v[r, sl] = rows_v[r, sl] * y
            return 0

        lax.fori_loop(0, CHUNK, row_body, 0)
        pltpu.async_copy(
            rows_v, out_hbm.at[pl.ds(base + ci * CHUNK, CHUNK)], osem
        ).wait()
        return 0

    lax.fori_loop(0, NCHUNK, chunk_body, 0)


def kernel(positions, positional_embeddings):
    idx = positions.reshape(-1).astype(jnp.int32)
    out = _gather_normalize(idx, positional_embeddings)
    return out.reshape(positions.shape + (D,))


# speedup vs baseline: 1.8827x; 1.7924x over previous
import functools

import jax
import jax.numpy as jnp
from jax import lax
from jax.experimental import pallas as pl
from jax.experimental.pallas import tpu as pltpu
from jax.experimental.pallas import tpu_sc as plsc

N_TOTAL = 32768
D = 1024
LANES = 16
NC, NS = 2, 16
NW = NC * NS
B_PER_W = N_TOTAL // NW
CHUNK = 16
NBUF = 4
NCHUNK = B_PER_W // CHUNK

_mesh = plsc.VectorSubcoreMesh(core_axis_name="c", subcore_axis_name="s")

_GATHER_DNUMS = lax.GatherDimensionNumbers(
    offset_dims=(), collapsed_slice_dims=(0,), start_index_map=(0,)
)


def _lane_shuffle(x, perm):
    return lax.gather(
        x, perm[:, None], _GATHER_DNUMS, (1,),
        mode=lax.GatherScatterMode.PROMISE_IN_BOUNDS,
    )


def _lane_sum(x):
    for sh in (8, 4, 2, 1):
        perm = jnp.arange(LANES, dtype=jnp.int32) ^ sh
        x = x + _lane_shuffle(x, perm)
    return x


def _rsqrt16(t):
    t = jnp.maximum(t, jnp.float32(1e-24))
    i = lax.bitcast_convert_type(t, jnp.int32)
    y = lax.bitcast_convert_type(jnp.int32(0x5F3759DF) - (i >> 1), jnp.float32)
    for _ in range(3):
        y = y * (jnp.float32(1.5) - jnp.float32(0.5) * t * y * y)
    return y


@functools.partial(
    pl.kernel,
    mesh=_mesh,
    out_type=jax.ShapeDtypeStruct((N_TOTAL, D), jnp.float32),
    scratch_types=[
        pltpu.VMEM((B_PER_W,), jnp.int32),
    ] + [pltpu.VMEM((CHUNK, D), jnp.float32)] * NBUF
      + [pltpu.SemaphoreType.DMA] * (2 * NBUF),
)
def _gather_normalize(idx_hbm, table_hbm, out_hbm, idx_v, *bufs_and_sems):
    rows = bufs_and_sems[:NBUF]
    gsem = bufs_and_sems[NBUF:2 * NBUF]
    osem = bufs_and_sems[2 * NBUF:]

    wid = lax.axis_index("s") * NC + lax.axis_index("c")
    base = wid * B_PER_W
    pltpu.sync_copy(idx_hbm.at[pl.ds(base, B_PER_W)], idx_v)

    def start_gather(j, b):
        pltpu.async_copy(
            table_hbm.at[idx_v.at[pl.ds(j * CHUNK, CHUNK)]], rows[b], gsem[b]
        )

    def start_scatter(j, b):
        pltpu.async_copy(
            rows[b], out_hbm.at[pl.ds(base + j * CHUNK, CHUNK)], osem[b]
        )

    def wait_gather(b):
        pltpu.make_async_copy(
            table_hbm.at[idx_v.at[pl.ds(0, CHUNK)]], rows[b], gsem[b]
        ).wait()

    def wait_scatter(b):
        pltpu.make_async_copy(
            rows[b], out_hbm.at[pl.ds(0, CHUNK)], osem[b]
        ).wait()

    def normalize_chunk(buf):
        def row_body(r, _):
            accs = [jnp.zeros((LANES,), jnp.float32) for _ in range(4)]
            for j in range(D // LANES):
                s = buf[r, pl.ds(j * LANES, LANES)]
                accs[j % 4] = accs[j % 4] + s * s
            acc = (accs[0] + accs[1]) + (accs[2] + accs[3])
            y = _rsqrt16(_lane_sum(acc))
            for j in range(D // LANES):
                sl = pl.ds(j * LANES, LANES)
                buf[r, sl] = buf[r, sl] * y
            return 0

        lax.fori_loop(0, CHUNK, row_body, 0)

    start_gather(0, 0)
    start_gather(1, 1)

    def group_body(g, _):
        for b in range(NBUF):
            i = g * NBUF + b
            wait_gather(b)
            normalize_chunk(rows[b])
            start_scatter(i, b)
            j = i + 2
            bj = (b + 2) % NBUF

            @pl.when(j < NCHUNK)
            def _():
                @pl.when(j >= NBUF)
                def _():
                    wait_scatter(bj)
                start_gather(j, bj)
        return 0

    lax.fori_loop(0, NCHUNK // NBUF, group_body, 0)

    for b in range(NBUF):
        wait_scatter(b)


def kernel(positions, positional_embeddings):
    idx = positions.reshape(-1).astype(jnp.int32)
    out = _gather_normalize(idx, positional_embeddings)
    return out.reshape(positions.shape + (D,))


# speedup vs baseline: 2.1991x; 1.1681x over previous
import functools

import jax
import jax.numpy as jnp
from jax import lax
from jax.experimental import pallas as pl
from jax.experimental.pallas import tpu as pltpu
from jax.experimental.pallas import tpu_sc as plsc

N_TOTAL = 32768
D = 1024
LANES = 16
NC, NS = 2, 16
NW = NC * NS
B_PER_W = N_TOTAL // NW
CHUNK = 16
NBUF = 4
NCHUNK = B_PER_W // CHUNK

_mesh = plsc.VectorSubcoreMesh(core_axis_name="c", subcore_axis_name="s")

_GATHER_DNUMS = lax.GatherDimensionNumbers(
    offset_dims=(), collapsed_slice_dims=(0,), start_index_map=(0,)
)


def _lane_shuffle(x, perm):
    return lax.gather(
        x, perm[:, None], _GATHER_DNUMS, (1,),
        mode=lax.GatherScatterMode.PROMISE_IN_BOUNDS,
    )


def _lane_sum(x):
    for sh in (8, 4, 2, 1):
        perm = jnp.arange(LANES, dtype=jnp.int32) ^ sh
        x = x + _lane_shuffle(x, perm)
    return x


def _rsqrt16(t):
    t = jnp.maximum(t, jnp.float32(1e-24))
    i = lax.bitcast_convert_type(t, jnp.int32)
    y = lax.bitcast_convert_type(jnp.int32(0x5F3759DF) - (i >> 1), jnp.float32)
    for _ in range(3):
        y = y * (jnp.float32(1.5) - jnp.float32(0.5) * t * y * y)
    return y


@functools.partial(
    pl.kernel,
    mesh=_mesh,
    out_type=jax.ShapeDtypeStruct((N_TOTAL, D), jnp.float32),
    scratch_types=[
        pltpu.VMEM((B_PER_W,), jnp.int32),
    ] + [pltpu.VMEM((CHUNK, D), jnp.float32)] * NBUF
      + [pltpu.SemaphoreType.DMA] * (2 * NBUF),
)
def _gather_normalize(idx_hbm, table_hbm, out_hbm, idx_v, *bufs_and_sems):
    rows = bufs_and_sems[:NBUF]
    gsem = bufs_and_sems[NBUF:2 * NBUF]
    osem = bufs_and_sems[2 * NBUF:]

    wid = lax.axis_index("s") * NC + lax.axis_index("c")
    base = wid * B_PER_W
    pltpu.sync_copy(idx_hbm.at[pl.ds(base, B_PER_W)], idx_v)

    def start_gather(j, b):
        pltpu.async_copy(
            table_hbm.at[idx_v.at[pl.ds(j * CHUNK, CHUNK)]], rows[b], gsem[b]
        )

    def start_scatter(j, b):
        pltpu.async_copy(
            rows[b], out_hbm.at[pl.ds(base + j * CHUNK, CHUNK)], osem[b]
        )

    def wait_gather(b):
        pltpu.make_async_copy(
            table_hbm.at[idx_v.at[pl.ds(0, CHUNK)]], rows[b], gsem[b]
        ).wait()

    def wait_scatter(b):
        pltpu.make_async_copy(
            rows[b], out_hbm.at[pl.ds(0, CHUNK)], osem[b]
        ).wait()

    lane_iota = jnp.arange(LANES, dtype=jnp.int32)

    def normalize_chunk(buf):
        def p1(r, z):
            accs = [jnp.zeros((LANES,), jnp.float32) for _ in range(4)]
            for j in range(D // LANES):
                s = buf[r, pl.ds(j * LANES, LANES)]
                accs[j % 4] = accs[j % 4] + s * s
            acc = _lane_sum((accs[0] + accs[1]) + (accs[2] + accs[3]))
            return jnp.where(lane_iota == r, acc, z)

        z = lax.fori_loop(0, CHUNK, p1, jnp.zeros((LANES,), jnp.float32))
        y = _rsqrt16(z)

        def p2(r, _):
            s = _lane_shuffle(y, jnp.full((LANES,), r, jnp.int32))
            for j in range(D // LANES):
                sl = pl.ds(j * LANES, LANES)
                buf[r, sl] = buf[r, sl] * s
            return 0

        lax.fori_loop(0, CHUNK, p2, 0)

    start_gather(0, 0)
    start_gather(1, 1)

    def group_body(g, _):
        for b in range(NBUF):
            i = g * NBUF + b
            wait_gather(b)
            normalize_chunk(rows[b])
            start_scatter(i, b)
            j = i + 2
            bj = (b + 2) % NBUF

            @pl.when(j < NCHUNK)
            def _():
                @pl.when(j >= NBUF)
                def _():
                    wait_scatter(bj)
                start_gather(j, bj)
        return 0

    lax.fori_loop(0, NCHUNK // NBUF, group_body, 0)

    for b in range(NBUF):
        wait_scatter(b)


def kernel(positions, positional_embeddings):
    idx = positions.reshape(-1).astype(jnp.int32)
    out = _gather_normalize(idx, positional_embeddings)
    return out.reshape(positions.shape + (D,))
